# baseline (speedup 1.0000x reference)
import functools

import jax
import jax.numpy as jnp
from jax import lax
from jax.experimental import pallas as pl
from jax.experimental.pallas import tpu as pltpu
from jax.experimental.pallas import tpu_sc as plsc

_L = 16


@functools.lru_cache(maxsize=None)
def _make_scan(n_rows, n_cols, nc=2, ns=16, chunk_rows=8):
    nw = nc * ns
    rows_per_w = n_rows // nw
    n_chunks = rows_per_w // chunk_rows
    chunk_elems = chunk_rows * n_cols
    vregs_per_row = n_cols // _L
    mesh = plsc.VectorSubcoreMesh(core_axis_name="c", subcore_axis_name="s")

    @functools.partial(
        pl.kernel,
        out_type=jax.ShapeDtypeStruct((n_rows * n_cols,), jnp.float32),
        mesh=mesh,
        scratch_types=[pltpu.VMEM((chunk_elems,), jnp.float32)],
        compiler_params=pltpu.CompilerParams(needs_layout_passes=False),
    )
    def scan_k(x_hbm, out_hbm, buf):
        wid = lax.axis_index("s") * nc + lax.axis_index("c")
        base = wid * rows_per_w * n_cols

        def chunk_body(ci, _):
            off = base + ci * chunk_elems
            pltpu.sync_copy(x_hbm.at[pl.ds(off, chunk_elems)], buf)

            def row_body(r, _):
                row_off = r * n_cols

                def vec_body(j, carry):
                    o = row_off + j * _L
                    v = buf[pl.ds(o, _L)]
                    buf[pl.ds(o, _L)] = plsc.cumsum(v) + carry
                    return carry + jnp.sum(v)

                lax.fori_loop(0, vregs_per_row, vec_body, jnp.float32(0.0))
                return 0

            lax.fori_loop(0, chunk_rows, row_body, 0)
            pltpu.sync_copy(buf, out_hbm.at[pl.ds(off, chunk_elems)])
            return 0

        lax.fori_loop(0, n_chunks, chunk_body, 0)

    return scan_k


def kernel(x):
    n_rows, n_cols = x.shape
    scan_k = _make_scan(n_rows, n_cols)
    return scan_k(x.reshape(-1)).reshape(n_rows, n_cols)


# speedup vs baseline: 2.0806x; 2.0806x over previous
import functools

import jax
import jax.numpy as jnp
from jax import lax
from jax.experimental import pallas as pl
from jax.experimental.pallas import tpu as pltpu
from jax.experimental.pallas import tpu_sc as plsc

_L = 16


@functools.lru_cache(maxsize=None)
def _make_scan(n_rows, n_cols, nc=2, ns=16, chunk_rows=8):
    nw = nc * ns
    rows_per_w = n_rows // nw
    n_chunks = rows_per_w // chunk_rows
    chunk_elems = chunk_rows * n_cols
    vregs_per_row = n_cols // _L
    mesh = plsc.VectorSubcoreMesh(core_axis_name="c", subcore_axis_name="s")

    @functools.partial(
        pl.kernel,
        out_type=jax.ShapeDtypeStruct((n_rows * n_cols,), jnp.float32),
        mesh=mesh,
        scratch_types=[pltpu.VMEM((chunk_elems,), jnp.float32)],
        compiler_params=pltpu.CompilerParams(needs_layout_passes=False),
    )
    def scan_k(x_hbm, out_hbm, buf):
        wid = lax.axis_index("s") * nc + lax.axis_index("c")
        base = wid * rows_per_w * n_cols

        def chunk_body(ci, _):
            off = base + ci * chunk_elems
            pltpu.sync_copy(x_hbm.at[pl.ds(off, chunk_elems)], buf)

            def row_body(r, _):
                row_off = r * n_cols

                def vec_body(j, carry):
                    o = row_off + j * _L
                    v = buf[pl.ds(o, _L)]
                    buf[pl.ds(o, _L)] = plsc.cumsum(v) + carry
                    return carry + jnp.sum(v)

                lax.fori_loop(0, vregs_per_row, vec_body, jnp.float32(0.0),
                              unroll=8)
                return 0

            lax.fori_loop(0, chunk_rows, row_body, 0)
            pltpu.sync_copy(buf, out_hbm.at[pl.ds(off, chunk_elems)])
            return 0

        lax.fori_loop(0, n_chunks, chunk_body, 0)

    return scan_k


def kernel(x):
    n_rows, n_cols = x.shape
    scan_k = _make_scan(n_rows, n_cols)
    return scan_k(x.reshape(-1)).reshape(n_rows, n_cols)


# speedup vs baseline: 2.2698x; 1.0909x over previous
import functools

import jax
import jax.numpy as jnp
from jax import lax
from jax.experimental import pallas as pl
from jax.experimental.pallas import tpu as pltpu
from jax.experimental.pallas import tpu_sc as plsc

_L = 16


@functools.lru_cache(maxsize=None)
def _make_scan(n_rows, n_cols, nc=2, ns=16, chunk_rows=8):
    nw = nc * ns
    rows_per_w = n_rows // nw
    n_chunks = rows_per_w // chunk_rows
    chunk_elems = chunk_rows * n_cols
    vregs_per_row = n_cols // _L
    mesh = plsc.VectorSubcoreMesh(core_axis_name="c", subcore_axis_name="s")

    @functools.partial(
        pl.kernel,
        out_type=jax.ShapeDtypeStruct((n_rows * n_cols,), jnp.float32),
        mesh=mesh,
        scratch_types=[pltpu.VMEM((chunk_elems,), jnp.float32)],
        compiler_params=pltpu.CompilerParams(needs_layout_passes=False),
    )
    def scan_k(x_hbm, out_hbm, buf):
        wid = lax.axis_index("s") * nc + lax.axis_index("c")
        base = wid * rows_per_w * n_cols

        def chunk_body(ci, _):
            off = base + ci * chunk_elems
            pltpu.sync_copy(x_hbm.at[pl.ds(off, chunk_elems)], buf)

            idx15 = jnp.full((_L,), _L - 1, jnp.int32)

            def row_body(r, _):
                row_off = r * n_cols

                def vec_body(j, carry):
                    o = row_off + j * _L
                    v = buf[pl.ds(o, _L)]
                    s = plsc.cumsum(v)
                    buf[pl.ds(o, _L)] = s + carry
                    return carry + s.at[idx15].get(mode="promise_in_bounds")

                lax.fori_loop(0, vregs_per_row, vec_body,
                              jnp.zeros((_L,), jnp.float32), unroll=8)
                return 0

            lax.fori_loop(0, chunk_rows, row_body, 0)
            pltpu.sync_copy(buf, out_hbm.at[pl.ds(off, chunk_elems)])
            return 0

        lax.fori_loop(0, n_chunks, chunk_body, 0)

    return scan_k


def kernel(x):
    n_rows, n_cols = x.shape
    scan_k = _make_scan(n_rows, n_cols)
    return scan_k(x.reshape(-1)).reshape(n_rows, n_cols)


# speedup vs baseline: 2.5877x; 1.1400x over previous
import functools

import jax
import jax.numpy as jnp
from jax import lax
from jax.experimental import pallas as pl
from jax.experimental.pallas import tpu as pltpu
from jax.experimental.pallas import tpu_sc as plsc

_L = 16


@functools.lru_cache(maxsize=None)
def _make_scan(n_rows, n_cols, nc=2, ns=16, chunk_rows=8):
    nw = nc * ns
    rows_per_w = n_rows // nw
    n_chunks = rows_per_w // chunk_rows
    vregs_per_row = n_cols // _L
    mesh = plsc.VectorSubcoreMesh(core_axis_name="c", subcore_axis_name="s")

    @functools.partial(
        pl.kernel,
        out_type=jax.ShapeDtypeStruct((n_rows, n_cols), jnp.float32),
        mesh=mesh,
        scratch_types=[pltpu.VMEM((n_cols,), jnp.float32)],
        compiler_params=pltpu.CompilerParams(needs_layout_passes=False),
    )
    def scan_k(x_hbm, out_hbm, buf):
        wid = lax.axis_index("s") * nc + lax.axis_index("c")
        row_base = wid * rows_per_w
        idx15 = jnp.full((_L,), _L - 1, jnp.int32)

        def row_body(r, _):
            pltpu.sync_copy(x_hbm.at[row_base + r], buf)

            def vec_body(j, carry):
                o = j * _L
                v = buf[pl.ds(o, _L)]
                s = plsc.cumsum(v)
                buf[pl.ds(o, _L)] = s + carry
                return carry + s.at[idx15].get(mode="promise_in_bounds")

            lax.fori_loop(0, vregs_per_row, vec_body,
                          jnp.zeros((_L,), jnp.float32), unroll=8)
            pltpu.sync_copy(buf, out_hbm.at[row_base + r])
            return 0

        lax.fori_loop(0, rows_per_w, row_body, 0)

    return scan_k


def kernel(x):
    n_rows, n_cols = x.shape
    scan_k = _make_scan(n_rows, n_cols)
    return scan_k(x)
